# Initial kernel scaffold; baseline (speedup 1.0000x reference)
import jax
import jax.numpy as jnp
from jax.experimental import pallas as pl


def kernel(x, edge_index, W1l, b1, W1r, W2l, b2, W2r):
    raise NotImplementedError("write your pallas kernel here")


# speedup vs baseline: 4.7393x; 4.7393x over previous
import functools

import jax
import jax.numpy as jnp
from jax import lax
from jax.experimental import pallas as pl
from jax.experimental.pallas import tpu as pltpu
from jax.experimental.pallas import tpu_sc as plsc

N = 10000
E = 320000
D = 128
NP = 10240
NC = 2
NS = 16
NW = NC * NS
EPW = E // NW
CH = 80
NCH = EPW // CH
RPS = NP // NS


def _edge_body(p_hbm, src_hbm, dst_hbm, out_hbm, src_v, dst_v, rows_v,
               acc_sh, sem):
  c = lax.axis_index("c")
  s = lax.axis_index("s")
  wid = c * NS + s
  base = wid * EPW

  zeros16 = jnp.zeros((16,), jnp.float32)

  def zero_row(i, _):
    for j in range(D // 16):
      rows_v[i, pl.ds(j * 16, 16)] = zeros16
    return 0

  lax.fori_loop(0, CH, zero_row, 0)
  row0 = s * RPS
  for i in range(RPS // CH):
    pltpu.sync_copy(rows_v, acc_sh.at[pl.ds(row0 + i * CH, CH)])
  plsc.subcore_barrier()

  def chunk(i, _):
    start = pl.multiple_of(base + i * CH, 8)
    pltpu.sync_copy(src_hbm.at[pl.ds(start, CH)], src_v)
    pltpu.sync_copy(dst_hbm.at[pl.ds(start, CH)], dst_v)
    pltpu.async_copy(p_hbm.at[src_v], rows_v, sem).wait()
    pltpu.sync_copy(rows_v, acc_sh.at[dst_v], add=True)
    return 0

  lax.fori_loop(0, NCH, chunk, 0)
  plsc.subcore_barrier()

  out_row = c * NP + s * RPS
  pltpu.sync_copy(acc_sh.at[pl.ds(s * RPS, RPS)],
                  out_hbm.at[pl.ds(out_row, RPS)])


_edge_pass = pl.kernel(
    _edge_body,
    out_type=jax.ShapeDtypeStruct((NC * NP, D), jnp.float32),
    mesh=plsc.VectorSubcoreMesh(core_axis_name="c", subcore_axis_name="s"),
    scratch_types=[
        pltpu.VMEM((CH,), jnp.int32),
        pltpu.VMEM((CH,), jnp.int32),
        pltpu.VMEM((CH, D), jnp.float32),
        pltpu.VMEM_SHARED((NP, D), jnp.float32),
        pltpu.SemaphoreType.DMA,
    ],
)


_BM = 2048
_EK = 2560


def _count_body(dst_ref, cnt_ref):
  d = dst_ref[0]
  hi = d // 128
  lo = d - hi * 128
  ihi = lax.broadcasted_iota(jnp.int32, (_EK, 128), 1)
  oh_hi = jnp.where(hi == ihi, 1.0, 0.0).astype(jnp.bfloat16)
  oh_lo = jnp.where(lo == ihi, 1.0, 0.0).astype(jnp.bfloat16)
  part = lax.dot_general(oh_hi, oh_lo, (((0,), (0,)), ((), ())),
                         preferred_element_type=jnp.float32)

  @pl.when(pl.program_id(0) == 0)
  def _():
    cnt_ref[...] = jnp.zeros_like(cnt_ref)

  cnt_ref[...] += part


def _count(dst3):
  return pl.pallas_call(
      _count_body,
      grid=(E // _EK,),
      in_specs=[pl.BlockSpec((1, _EK, 1), lambda i: (i, 0, 0))],
      out_specs=pl.BlockSpec((128, 128), lambda i: (0, 0)),
      out_shape=jax.ShapeDtypeStruct((128, 128), jnp.float32),
  )(dst3)


def _proj_body(h_ref, wl_ref, wr_ref, b_ref, p_ref, q_ref):
  h = h_ref[...]
  p_ref[...] = jnp.dot(h, wl_ref[...], preferred_element_type=jnp.float32)
  q_ref[...] = (jnp.dot(h, wr_ref[...], preferred_element_type=jnp.float32)
                + b_ref[...])


def _proj(h, wlT, wrT, b):
  return pl.pallas_call(
      _proj_body,
      grid=(NP // _BM,),
      in_specs=[
          pl.BlockSpec((_BM, D), lambda i: (i, 0)),
          pl.BlockSpec((D, D), lambda i: (0, 0)),
          pl.BlockSpec((D, D), lambda i: (0, 0)),
          pl.BlockSpec((1, D), lambda i: (0, 0)),
      ],
      out_specs=[
          pl.BlockSpec((_BM, D), lambda i: (i, 0)),
          pl.BlockSpec((_BM, D), lambda i: (i, 0)),
      ],
      out_shape=[
          jax.ShapeDtypeStruct((NP, D), jnp.float32),
          jax.ShapeDtypeStruct((NP, D), jnp.float32),
      ],
  )(h, wlT, wrT, b)


def _combine_body(part_ref, cnt_ref, q_ref, o_ref, *, softmax):
  ssum = part_ref[0] + part_ref[1]
  c = cnt_ref[...]
  h = jnp.maximum(ssum / jnp.maximum(c, 1.0) + q_ref[...], 0.0)
  if softmax:
    m = jnp.max(h, axis=1, keepdims=True)
    lse = jnp.log(jnp.sum(jnp.exp(h - m), axis=1, keepdims=True)) + m
    h = h - lse
  o_ref[...] = h


def _combine(part, cnt, q, softmax):
  return pl.pallas_call(
      functools.partial(_combine_body, softmax=softmax),
      grid=(NP // _BM,),
      in_specs=[
          pl.BlockSpec((2, _BM, D), lambda i: (0, i, 0)),
          pl.BlockSpec((_BM, 1), lambda i: (i, 0)),
          pl.BlockSpec((_BM, D), lambda i: (i, 0)),
      ],
      out_specs=pl.BlockSpec((_BM, D), lambda i: (i, 0)),
      out_shape=jax.ShapeDtypeStruct((NP, D), jnp.float32),
  )(part, cnt, q)


def kernel(x, edge_index, W1l, b1, W1r, W2l, b2, W2r):
  src = edge_index[0]
  dst = edge_index[1]
  xp = jnp.pad(x, ((0, NP - N), (0, 0)))

  cnt = _count(dst.reshape(E // _EK, _EK, 1)).reshape(16384)[:NP]
  cnt = cnt.reshape(NP, 1)

  p1, q1 = _proj(xp, W1l.T, W1r.T, b1[None])
  acc1 = _edge_pass(p1, src, dst).reshape(NC, NP, D)
  h1 = _combine(acc1, cnt, q1, softmax=False)

  p2, q2 = _proj(h1, W2l.T, W2r.T, b2[None])
  acc2 = _edge_pass(p2, src, dst).reshape(NC, NP, D)
  h2 = _combine(acc2, cnt, q2, softmax=True)
  return h2[:N]


# speedup vs baseline: 6.1508x; 1.2978x over previous
import functools

import jax
import jax.numpy as jnp
from jax import lax
from jax.experimental import pallas as pl
from jax.experimental.pallas import tpu as pltpu
from jax.experimental.pallas import tpu_sc as plsc

N = 10000
E = 320000
D = 128
NP = 10240
NC = 2
NS = 16
NW = NC * NS
EPW = E // NW
CH = 80
NCH = EPW // CH
RPS = NP // NS


NB = 3
GD = 2
NBI = 6


def _edge_body(p_hbm, ei_hbm, out_hbm, ei_v, rows_v, acc_sh, sem_i, sem_g,
               sem_s):
  c = lax.axis_index("c")
  s = lax.axis_index("s")
  wid = c * NS + s

  zeros16 = jnp.zeros((16,), jnp.float32)

  def zero_row(i, _):
    for j in range(D // 16):
      rows_v[0, i, pl.ds(j * 16, 16)] = zeros16
    return 0

  lax.fori_loop(0, CH, zero_row, 0)
  row0 = s * RPS
  for i in range(RPS // CH):
    pltpu.sync_copy(rows_v.at[0], acc_sh.at[pl.ds(row0 + i * CH, CH)])
  plsc.subcore_barrier()

  def idx_start(f):
    bi = f % NBI
    pltpu.async_copy(ei_hbm.at[wid, f], ei_v.at[bi], sem_i.at[bi])

  def idx_wait(f):
    bi = f % NBI
    pltpu.make_async_copy(ei_hbm.at[wid, f], ei_v.at[bi],
                          sem_i.at[bi]).wait()

  def gather_start(f):
    b = f % NB
    pltpu.async_copy(p_hbm.at[ei_v.at[f % NBI, 0]], rows_v.at[b],
                     sem_g.at[b])

  def gather_wait(j):
    b = j % NB
    pltpu.make_async_copy(p_hbm.at[ei_v.at[j % NBI, 0]], rows_v.at[b],
                          sem_g.at[b]).wait()

  def scatter_start(j):
    b = j % NB
    pltpu.async_copy(rows_v.at[b], acc_sh.at[ei_v.at[j % NBI, 1]],
                     sem_s.at[b], add=True)

  def scatter_wait(j):
    b = j % NB
    pltpu.make_async_copy(rows_v.at[b], acc_sh.at[ei_v.at[j % NBI, 1]],
                          sem_s.at[b]).wait()

  for f in range(GD + 1):
    idx_start(f)
  for f in range(GD):
    idx_wait(f)
    gather_start(f)

  def chunk(j, _):
    gather_wait(j)
    scatter_start(j)
    f = j + GD

    @pl.when(f < NCH)
    def _():
      idx_wait(f)
      @pl.when(f >= NB)
      def _():
        scatter_wait(f - NB)

      gather_start(f)

    fx = j + GD + 1

    @pl.when(fx < NCH)
    def _():
      idx_start(fx)

    return 0

  lax.fori_loop(0, NCH, chunk, 0)
  for j in range(NCH - GD - 1, NCH):
    scatter_wait(j)
  plsc.subcore_barrier()

  out_row = c * NP + s * RPS
  pltpu.sync_copy(acc_sh.at[pl.ds(s * RPS, RPS)],
                  out_hbm.at[pl.ds(out_row, RPS)])


_edge_pass = pl.kernel(
    _edge_body,
    out_type=jax.ShapeDtypeStruct((NC * NP, D), jnp.float32),
    mesh=plsc.VectorSubcoreMesh(core_axis_name="c", subcore_axis_name="s"),
    scratch_types=[
        pltpu.VMEM((NBI, 2, CH), jnp.int32),
        pltpu.VMEM((NB, CH, D), jnp.float32),
        pltpu.VMEM_SHARED((NP, D), jnp.float32),
        pltpu.SemaphoreType.DMA((NBI,)),
        pltpu.SemaphoreType.DMA((NB,)),
        pltpu.SemaphoreType.DMA((NB,)),
    ],
)


_BM = 2048
_EK = 2560


def _count_body(dst_ref, cnt_ref):
  d = dst_ref[0]
  hi = d // 128
  lo = d - hi * 128
  ihi = lax.broadcasted_iota(jnp.int32, (_EK, 128), 1)
  oh_hi = jnp.where(hi == ihi, 1.0, 0.0).astype(jnp.bfloat16)
  oh_lo = jnp.where(lo == ihi, 1.0, 0.0).astype(jnp.bfloat16)
  part = lax.dot_general(oh_hi, oh_lo, (((0,), (0,)), ((), ())),
                         preferred_element_type=jnp.float32)

  @pl.when(pl.program_id(0) == 0)
  def _():
    cnt_ref[...] = jnp.zeros_like(cnt_ref)

  cnt_ref[...] += part


def _count(dst3):
  return pl.pallas_call(
      _count_body,
      grid=(E // _EK,),
      in_specs=[pl.BlockSpec((1, _EK, 1), lambda i: (i, 0, 0))],
      out_specs=pl.BlockSpec((128, 128), lambda i: (0, 0)),
      out_shape=jax.ShapeDtypeStruct((128, 128), jnp.float32),
  )(dst3)


def _proj_body(h_ref, wl_ref, wr_ref, b_ref, p_ref, q_ref):
  h = h_ref[...]
  p_ref[...] = jnp.dot(h, wl_ref[...], preferred_element_type=jnp.float32)
  q_ref[...] = (jnp.dot(h, wr_ref[...], preferred_element_type=jnp.float32)
                + b_ref[...])


def _proj(h, wlT, wrT, b):
  return pl.pallas_call(
      _proj_body,
      grid=(NP // _BM,),
      in_specs=[
          pl.BlockSpec((_BM, D), lambda i: (i, 0)),
          pl.BlockSpec((D, D), lambda i: (0, 0)),
          pl.BlockSpec((D, D), lambda i: (0, 0)),
          pl.BlockSpec((1, D), lambda i: (0, 0)),
      ],
      out_specs=[
          pl.BlockSpec((_BM, D), lambda i: (i, 0)),
          pl.BlockSpec((_BM, D), lambda i: (i, 0)),
      ],
      out_shape=[
          jax.ShapeDtypeStruct((NP, D), jnp.float32),
          jax.ShapeDtypeStruct((NP, D), jnp.float32),
      ],
  )(h, wlT, wrT, b)


def _combine_body(part_ref, cnt_ref, q_ref, o_ref, *, softmax):
  ssum = part_ref[0] + part_ref[1]
  c = cnt_ref[...]
  h = jnp.maximum(ssum / jnp.maximum(c, 1.0) + q_ref[...], 0.0)
  if softmax:
    m = jnp.max(h, axis=1, keepdims=True)
    lse = jnp.log(jnp.sum(jnp.exp(h - m), axis=1, keepdims=True)) + m
    h = h - lse
  o_ref[...] = h


def _combine(part, cnt, q, softmax):
  return pl.pallas_call(
      functools.partial(_combine_body, softmax=softmax),
      grid=(NP // _BM,),
      in_specs=[
          pl.BlockSpec((2, _BM, D), lambda i: (0, i, 0)),
          pl.BlockSpec((_BM, 1), lambda i: (i, 0)),
          pl.BlockSpec((_BM, D), lambda i: (i, 0)),
      ],
      out_specs=pl.BlockSpec((_BM, D), lambda i: (i, 0)),
      out_shape=jax.ShapeDtypeStruct((NP, D), jnp.float32),
  )(part, cnt, q)


def kernel(x, edge_index, W1l, b1, W1r, W2l, b2, W2r):
  src = edge_index[0]
  dst = edge_index[1]
  xp = jnp.pad(x, ((0, NP - N), (0, 0)))
  ei_re = jnp.stack([src.reshape(NW, NCH, CH), dst.reshape(NW, NCH, CH)],
                    axis=2)

  cnt = _count(dst.reshape(E // _EK, _EK, 1)).reshape(16384)[:NP]
  cnt = cnt.reshape(NP, 1)

  p1, q1 = _proj(xp, W1l.T, W1r.T, b1[None])
  acc1 = _edge_pass(p1, ei_re).reshape(NC, NP, D)
  h1 = _combine(acc1, cnt, q1, softmax=False)

  p2, q2 = _proj(h1, W2l.T, W2r.T, b2[None])
  acc2 = _edge_pass(p2, ei_re).reshape(NC, NP, D)
  h2 = _combine(acc2, cnt, q2, softmax=True)
  return h2[:N]


# speedup vs baseline: 13.1278x; 2.1343x over previous
import functools

import jax
import jax.numpy as jnp
from jax import lax
from jax.experimental import pallas as pl
from jax.experimental.pallas import tpu as pltpu
from jax.experimental.pallas import tpu_sc as plsc

N = 10000
E = 320000
D = 128
NP = 10240
NC = 2
NS = 16
NW = NC * NS
EPW = E // NW
CH = 80
NCH = EPW // CH
RPS = NP // NS


NB = 3
GD = 2
NBI = 6


def _edge_body(p_hbm, ei_hbm, out_hbm, ei_v, rows_v, acc_sh, sem_i, sem_g,
               sem_s):
  c = lax.axis_index("c")
  s = lax.axis_index("s")
  wid = c * NS + s

  zeros16 = jnp.zeros((16,), jnp.float32)

  def zero_row(i, _):
    for j in range(D // 16):
      rows_v[0, i, pl.ds(j * 16, 16)] = zeros16
    return 0

  lax.fori_loop(0, CH, zero_row, 0)
  row0 = s * RPS
  for i in range(RPS // CH):
    pltpu.sync_copy(rows_v.at[0], acc_sh.at[pl.ds(row0 + i * CH, CH)])
  plsc.subcore_barrier()

  def idx_start(f):
    bi = f % NBI
    pltpu.async_copy(ei_hbm.at[wid, f], ei_v.at[bi], sem_i.at[bi])

  def idx_wait(f):
    bi = f % NBI
    pltpu.make_async_copy(ei_hbm.at[wid, f], ei_v.at[bi],
                          sem_i.at[bi]).wait()

  def gather_start(f):
    b = f % NB
    pltpu.async_copy(p_hbm.at[ei_v.at[f % NBI, 0]], rows_v.at[b],
                     sem_g.at[b])

  def gather_wait(j):
    b = j % NB
    pltpu.make_async_copy(p_hbm.at[ei_v.at[j % NBI, 0]], rows_v.at[b],
                          sem_g.at[b]).wait()

  def scatter_start(j):
    b = j % NB
    pltpu.async_copy(rows_v.at[b], acc_sh.at[ei_v.at[j % NBI, 1]],
                     sem_s.at[b], add=True)

  def scatter_wait(j):
    b = j % NB
    pltpu.make_async_copy(rows_v.at[b], acc_sh.at[ei_v.at[j % NBI, 1]],
                          sem_s.at[b]).wait()

  for f in range(GD + 1):
    idx_start(f)
  for f in range(GD):
    idx_wait(f)
    gather_start(f)

  def chunk(j, _):
    gather_wait(j)
    scatter_start(j)
    f = j + GD

    @pl.when(f < NCH)
    def _():
      idx_wait(f)
      @pl.when(f >= NB)
      def _():
        scatter_wait(f - NB)

      gather_start(f)

    fx = j + GD + 1

    @pl.when(fx < NCH)
    def _():
      idx_start(fx)

    return 0

  lax.fori_loop(0, NCH, chunk, 0)
  for j in range(NCH - GD - 1, NCH):
    scatter_wait(j)
  plsc.subcore_barrier()

  out_row = c * NP + s * RPS
  pltpu.sync_copy(acc_sh.at[pl.ds(s * RPS, RPS)],
                  out_hbm.at[pl.ds(out_row, RPS)])


_edge_pass = pl.kernel(
    _edge_body,
    out_type=jax.ShapeDtypeStruct((NC * NP, D), jnp.float32),
    mesh=plsc.VectorSubcoreMesh(core_axis_name="c", subcore_axis_name="s"),
    scratch_types=[
        pltpu.VMEM((NBI, 2, CH), jnp.int32),
        pltpu.VMEM((NB, CH, D), jnp.float32),
        pltpu.VMEM_SHARED((NP, D), jnp.float32),
        pltpu.SemaphoreType.DMA((NBI,)),
        pltpu.SemaphoreType.DMA((NB,)),
        pltpu.SemaphoreType.DMA((NB,)),
    ],
)


_BM = 2048
_EK = 6400
_NH = NP // 128


def _count_body(dst_ref, cnt_ref):
  d = dst_ref[0]
  hi = d // 128
  lo = d - hi * 128
  ihi = lax.broadcasted_iota(jnp.int32, (_NH, _EK), 0)
  ilo = lax.broadcasted_iota(jnp.int32, (128, _EK), 0)
  oh_hi = jnp.where(hi == ihi, 1.0, 0.0).astype(jnp.bfloat16)
  oh_lo = jnp.where(lo == ilo, 1.0, 0.0).astype(jnp.bfloat16)
  part = lax.dot_general(oh_hi, oh_lo, (((1,), (1,)), ((), ())),
                         preferred_element_type=jnp.float32)

  @pl.when(pl.program_id(0) == 0)
  def _():
    cnt_ref[...] = jnp.zeros_like(cnt_ref)

  cnt_ref[...] += part


def _count(dst3):
  return pl.pallas_call(
      _count_body,
      grid=(E // _EK,),
      in_specs=[pl.BlockSpec((1, 1, _EK), lambda i: (i, 0, 0))],
      out_specs=pl.BlockSpec((_NH, 128), lambda i: (0, 0)),
      out_shape=jax.ShapeDtypeStruct((_NH, 128), jnp.float32),
  )(dst3)


def _proj_body(h_ref, wl_ref, wr_ref, b_ref, p_ref, q_ref):
  h = h_ref[...]
  p_ref[...] = jnp.dot(h, wl_ref[...], preferred_element_type=jnp.float32)
  q_ref[...] = (jnp.dot(h, wr_ref[...], preferred_element_type=jnp.float32)
                + b_ref[...])


def _proj(h, wlT, wrT, b):
  return pl.pallas_call(
      _proj_body,
      grid=(NP // _BM,),
      in_specs=[
          pl.BlockSpec((_BM, D), lambda i: (i, 0)),
          pl.BlockSpec((D, D), lambda i: (0, 0)),
          pl.BlockSpec((D, D), lambda i: (0, 0)),
          pl.BlockSpec((1, D), lambda i: (0, 0)),
      ],
      out_specs=[
          pl.BlockSpec((_BM, D), lambda i: (i, 0)),
          pl.BlockSpec((_BM, D), lambda i: (i, 0)),
      ],
      out_shape=[
          jax.ShapeDtypeStruct((NP, D), jnp.float32),
          jax.ShapeDtypeStruct((NP, D), jnp.float32),
      ],
  )(h, wlT, wrT, b)


def _combine_body(part_ref, cnt_ref, q_ref, o_ref, *, softmax):
  ssum = part_ref[0] + part_ref[1]
  c = cnt_ref[...]
  h = jnp.maximum(ssum / jnp.maximum(c, 1.0) + q_ref[...], 0.0)
  if softmax:
    m = jnp.max(h, axis=1, keepdims=True)
    lse = jnp.log(jnp.sum(jnp.exp(h - m), axis=1, keepdims=True)) + m
    h = h - lse
  o_ref[...] = h


def _combine(part, cnt, q, softmax):
  return pl.pallas_call(
      functools.partial(_combine_body, softmax=softmax),
      grid=(NP // _BM,),
      in_specs=[
          pl.BlockSpec((2, _BM, D), lambda i: (0, i, 0)),
          pl.BlockSpec((_BM, 1), lambda i: (i, 0)),
          pl.BlockSpec((_BM, D), lambda i: (i, 0)),
      ],
      out_specs=pl.BlockSpec((_BM, D), lambda i: (i, 0)),
      out_shape=jax.ShapeDtypeStruct((NP, D), jnp.float32),
  )(part, cnt, q)


def kernel(x, edge_index, W1l, b1, W1r, W2l, b2, W2r):
  src = edge_index[0]
  dst = edge_index[1]
  xp = jnp.pad(x, ((0, NP - N), (0, 0)))
  ei_re = jnp.stack([src.reshape(NW, NCH, CH), dst.reshape(NW, NCH, CH)],
                    axis=2)

  cnt = _count(dst.reshape(E // _EK, 1, _EK)).reshape(NP, 1)

  p1, q1 = _proj(xp, W1l.T, W1r.T, b1[None])
  acc1 = _edge_pass(p1, ei_re).reshape(NC, NP, D)
  h1 = _combine(acc1, cnt, q1, softmax=False)

  p2, q2 = _proj(h1, W2l.T, W2r.T, b2[None])
  acc2 = _edge_pass(p2, ei_re).reshape(NC, NP, D)
  h2 = _combine(acc2, cnt, q2, softmax=True)
  return h2[:N]


# speedup vs baseline: 14.0630x; 1.0712x over previous
import functools

import jax
import jax.numpy as jnp
from jax import lax
from jax.experimental import pallas as pl
from jax.experimental.pallas import tpu as pltpu
from jax.experimental.pallas import tpu_sc as plsc

N = 10000
E = 320000
D = 128
NP = 10240
NC = 2
NS = 16
NW = NC * NS
EPW = E // NW
CH = 80
NCH = EPW // CH
RPS = NP // NS


NB = 4
GD = 2
NBI = 6


def _edge_body(p_hbm, src_hbm, dst_hbm, out_hbm, ei_v, rows_v, acc_sh, sem_i,
               sem_g, sem_s):
  c = lax.axis_index("c")
  s = lax.axis_index("s")
  wid = c * NS + s
  base = wid * EPW

  zeros16 = jnp.zeros((16,), jnp.float32)

  def zero_row(i, _):
    for j in range(D // 16):
      rows_v[0, i, pl.ds(j * 16, 16)] = zeros16
    return 0

  lax.fori_loop(0, CH, zero_row, 0)
  row0 = s * RPS
  for i in range(RPS // CH):
    pltpu.sync_copy(rows_v.at[0], acc_sh.at[pl.ds(row0 + i * CH, CH)])
  plsc.subcore_barrier()

  def idx_descs(f):
    start = pl.multiple_of(base + f * CH, 8)
    bi = f % NBI
    return (
        pltpu.make_async_copy(src_hbm.at[pl.ds(start, CH)], ei_v.at[bi, 0],
                              sem_i.at[bi]),
        pltpu.make_async_copy(dst_hbm.at[pl.ds(start, CH)], ei_v.at[bi, 1],
                              sem_i.at[bi]),
    )

  def idx_start(f):
    for dsc in idx_descs(f):
      dsc.start()

  def idx_wait(f):
    for dsc in idx_descs(f):
      dsc.wait()

  def gather_start(f):
    b = f % NB
    pltpu.async_copy(p_hbm.at[ei_v.at[f % NBI, 0]], rows_v.at[b],
                     sem_g.at[b])

  def gather_wait(j):
    b = j % NB
    pltpu.make_async_copy(p_hbm.at[ei_v.at[j % NBI, 0]], rows_v.at[b],
                          sem_g.at[b]).wait()

  def scatter_start(j):
    b = j % NB
    pltpu.async_copy(rows_v.at[b], acc_sh.at[ei_v.at[j % NBI, 1]],
                     sem_s.at[b], add=True)

  def scatter_wait(j):
    b = j % NB
    pltpu.make_async_copy(rows_v.at[b], acc_sh.at[ei_v.at[j % NBI, 1]],
                          sem_s.at[b]).wait()

  for f in range(GD + 1):
    idx_start(f)
  for f in range(GD):
    idx_wait(f)
    gather_start(f)

  def chunk(j, _):
    gather_wait(j)
    scatter_start(j)
    f = j + GD

    @pl.when(f < NCH)
    def _():
      idx_wait(f)
      @pl.when(f >= NB)
      def _():
        scatter_wait(f - NB)

      gather_start(f)

    fx = j + GD + 1

    @pl.when(fx < NCH)
    def _():
      idx_start(fx)

    return 0

  lax.fori_loop(0, NCH, chunk, 0)
  for j in range(NCH - GD - 1, NCH):
    scatter_wait(j)
  plsc.subcore_barrier()

  out_row = c * NP + s * RPS
  pltpu.sync_copy(acc_sh.at[pl.ds(s * RPS, RPS)],
                  out_hbm.at[pl.ds(out_row, RPS)])


_edge_pass = pl.kernel(
    _edge_body,
    out_type=jax.ShapeDtypeStruct((NC * NP, D), jnp.float32),
    mesh=plsc.VectorSubcoreMesh(core_axis_name="c", subcore_axis_name="s"),
    scratch_types=[
        pltpu.VMEM((NBI, 2, CH), jnp.int32),
        pltpu.VMEM((NB, CH, D), jnp.float32),
        pltpu.VMEM_SHARED((NP, D), jnp.float32),
        pltpu.SemaphoreType.DMA((NBI,)),
        pltpu.SemaphoreType.DMA((NB,)),
        pltpu.SemaphoreType.DMA((NB,)),
    ],
)


_BM = 2048
_EK = 6400
_NH = NP // 128


def _count_body(dst_ref, cnt_ref):
  d = dst_ref[0]
  hi = d // 128
  lo = d - hi * 128
  ihi = lax.broadcasted_iota(jnp.int32, (_NH, _EK), 0)
  ilo = lax.broadcasted_iota(jnp.int32, (128, _EK), 0)
  oh_hi = jnp.where(hi == ihi, 1.0, 0.0).astype(jnp.bfloat16)
  oh_lo = jnp.where(lo == ilo, 1.0, 0.0).astype(jnp.bfloat16)
  part = lax.dot_general(oh_hi, oh_lo, (((1,), (1,)), ((), ())),
                         preferred_element_type=jnp.float32)

  @pl.when(pl.program_id(0) == 0)
  def _():
    cnt_ref[...] = jnp.zeros_like(cnt_ref)

  cnt_ref[...] += part


def _count(dst3):
  return pl.pallas_call(
      _count_body,
      grid=(E // _EK,),
      in_specs=[pl.BlockSpec((1, 1, _EK), lambda i: (i, 0, 0))],
      out_specs=pl.BlockSpec((_NH, 128), lambda i: (0, 0)),
      out_shape=jax.ShapeDtypeStruct((_NH, 128), jnp.float32),
  )(dst3)


def _proj_body(h_ref, wl_ref, wr_ref, b_ref, p_ref, q_ref):
  h = h_ref[...]
  p_ref[...] = jnp.dot(h, wl_ref[...], preferred_element_type=jnp.float32)
  q_ref[...] = (jnp.dot(h, wr_ref[...], preferred_element_type=jnp.float32)
                + b_ref[...])


def _proj(h, wlT, wrT, b):
  return pl.pallas_call(
      _proj_body,
      grid=(NP // _BM,),
      in_specs=[
          pl.BlockSpec((_BM, D), lambda i: (i, 0)),
          pl.BlockSpec((D, D), lambda i: (0, 0)),
          pl.BlockSpec((D, D), lambda i: (0, 0)),
          pl.BlockSpec((1, D), lambda i: (0, 0)),
      ],
      out_specs=[
          pl.BlockSpec((_BM, D), lambda i: (i, 0)),
          pl.BlockSpec((_BM, D), lambda i: (i, 0)),
      ],
      out_shape=[
          jax.ShapeDtypeStruct((NP, D), jnp.float32),
          jax.ShapeDtypeStruct((NP, D), jnp.float32),
      ],
  )(h, wlT, wrT, b)


def _combine_body(part_ref, cnt_ref, q_ref, o_ref, *, softmax):
  ssum = part_ref[0] + part_ref[1]
  c = cnt_ref[...]
  h = jnp.maximum(ssum / jnp.maximum(c, 1.0) + q_ref[...], 0.0)
  if softmax:
    m = jnp.max(h, axis=1, keepdims=True)
    lse = jnp.log(jnp.sum(jnp.exp(h - m), axis=1, keepdims=True)) + m
    h = h - lse
  o_ref[...] = h


def _combine(part, cnt, q, softmax):
  return pl.pallas_call(
      functools.partial(_combine_body, softmax=softmax),
      grid=(NP // _BM,),
      in_specs=[
          pl.BlockSpec((2, _BM, D), lambda i: (0, i, 0)),
          pl.BlockSpec((_BM, 1), lambda i: (i, 0)),
          pl.BlockSpec((_BM, D), lambda i: (i, 0)),
      ],
      out_specs=pl.BlockSpec((_BM, D), lambda i: (i, 0)),
      out_shape=jax.ShapeDtypeStruct((NP, D), jnp.float32),
  )(part, cnt, q)


def kernel(x, edge_index, W1l, b1, W1r, W2l, b2, W2r):
  src = edge_index[0]
  dst = edge_index[1]
  xp = jnp.pad(x, ((0, NP - N), (0, 0)))

  cnt = _count(dst.reshape(E // _EK, 1, _EK)).reshape(NP, 1)

  p1, q1 = _proj(xp, W1l.T, W1r.T, b1[None])
  acc1 = _edge_pass(p1, src, dst).reshape(NC, NP, D)
  h1 = _combine(acc1, cnt, q1, softmax=False)

  p2, q2 = _proj(h1, W2l.T, W2r.T, b2[None])
  acc2 = _edge_pass(p2, src, dst).reshape(NC, NP, D)
  h2 = _combine(acc2, cnt, q2, softmax=True)
  return h2[:N]


# speedup vs baseline: 14.5903x; 1.0375x over previous
import functools

import jax
import jax.numpy as jnp
from jax import lax
from jax.experimental import pallas as pl
from jax.experimental.pallas import tpu as pltpu
from jax.experimental.pallas import tpu_sc as plsc

N = 10000
E = 320000
D = 128
NP = 10240
NC = 2
NS = 16
NW = NC * NS
EPW = E // NW
CH = 80
NCH = EPW // CH
RPS = NP // NS


NB = 4
GD = 2
NBI = 6


def _edge_body(p_hbm, src_hbm, dst_hbm, out_hbm, ei_v, rows_v, acc_sh, sem_i,
               sem_g, sem_s):
  c = lax.axis_index("c")
  s = lax.axis_index("s")
  wid = c * NS + s
  base = wid * EPW

  zeros16 = jnp.zeros((16,), jnp.float32)

  def zero_row(i, _):
    for j in range(D // 16):
      rows_v[0, i, pl.ds(j * 16, 16)] = zeros16
    return 0

  lax.fori_loop(0, CH, zero_row, 0)
  row0 = s * RPS
  for i in range(RPS // CH):
    pltpu.sync_copy(rows_v.at[0], acc_sh.at[pl.ds(row0 + i * CH, CH)])
  plsc.subcore_barrier()

  def idx_descs(f):
    start = pl.multiple_of(base + f * CH, 8)
    bi = f % NBI
    return (
        pltpu.make_async_copy(src_hbm.at[pl.ds(start, CH)], ei_v.at[bi, 0],
                              sem_i.at[bi]),
        pltpu.make_async_copy(dst_hbm.at[pl.ds(start, CH)], ei_v.at[bi, 1],
                              sem_i.at[bi]),
    )

  def idx_start(f):
    for dsc in idx_descs(f):
      dsc.start()

  def idx_wait(f):
    for dsc in idx_descs(f):
      dsc.wait()

  def gather_start(f):
    b = f % NB
    pltpu.async_copy(p_hbm.at[ei_v.at[f % NBI, 0]], rows_v.at[b],
                     sem_g.at[b])

  def gather_wait(j):
    b = j % NB
    pltpu.make_async_copy(p_hbm.at[ei_v.at[j % NBI, 0]], rows_v.at[b],
                          sem_g.at[b]).wait()

  def scatter_start(j):
    b = j % NB
    pltpu.async_copy(rows_v.at[b], acc_sh.at[ei_v.at[j % NBI, 1]],
                     sem_s.at[b], add=True)

  def scatter_wait(j):
    b = j % NB
    pltpu.make_async_copy(rows_v.at[b], acc_sh.at[ei_v.at[j % NBI, 1]],
                          sem_s.at[b]).wait()

  for f in range(GD + 1):
    idx_start(f)
  for f in range(GD):
    idx_wait(f)
    gather_start(f)

  def chunk(j, _):
    gather_wait(j)
    scatter_start(j)
    f = j + GD

    @pl.when(f < NCH)
    def _():
      idx_wait(f)
      @pl.when(f >= NB)
      def _():
        scatter_wait(f - NB)

      gather_start(f)

    fx = j + GD + 1

    @pl.when(fx < NCH)
    def _():
      idx_start(fx)

    return 0

  lax.fori_loop(0, NCH, chunk, 0)
  for j in range(NCH - GD - 1, NCH):
    scatter_wait(j)
  plsc.subcore_barrier()

  out_row = c * NP + s * RPS
  pltpu.sync_copy(acc_sh.at[pl.ds(s * RPS, RPS)],
                  out_hbm.at[pl.ds(out_row, RPS)])


_edge_pass = pl.kernel(
    _edge_body,
    out_type=jax.ShapeDtypeStruct((NC * NP, D), jnp.float32),
    mesh=plsc.VectorSubcoreMesh(core_axis_name="c", subcore_axis_name="s"),
    scratch_types=[
        pltpu.VMEM((NBI, 2, CH), jnp.int32),
        pltpu.VMEM((NB, CH, D), jnp.float32),
        pltpu.VMEM_SHARED((NP, D), jnp.float32),
        pltpu.SemaphoreType.DMA((NBI,)),
        pltpu.SemaphoreType.DMA((NB,)),
        pltpu.SemaphoreType.DMA((NB,)),
    ],
)


_BM = 2048
_EK = 6400
_NH = NP // 128


def _count_body(dst_ref, cnt_ref):
  d = dst_ref[1:2, :]
  hi = d // 128
  lo = d - hi * 128
  ihi = lax.broadcasted_iota(jnp.int32, (_NH, _EK), 0)
  ilo = lax.broadcasted_iota(jnp.int32, (128, _EK), 0)
  oh_hi = jnp.where(hi == ihi, 1.0, 0.0).astype(jnp.bfloat16)
  oh_lo = jnp.where(lo == ilo, 1.0, 0.0).astype(jnp.bfloat16)
  part = lax.dot_general(oh_hi, oh_lo, (((1,), (1,)), ((), ())),
                         preferred_element_type=jnp.float32)

  @pl.when(pl.program_id(0) == 0)
  def _():
    cnt_ref[...] = jnp.zeros_like(cnt_ref)

  cnt_ref[...] += part


def _count(edge_index):
  return pl.pallas_call(
      _count_body,
      grid=(E // _EK,),
      in_specs=[pl.BlockSpec((2, _EK), lambda i: (0, i))],
      out_specs=pl.BlockSpec((_NH, 128), lambda i: (0, 0)),
      out_shape=jax.ShapeDtypeStruct((_NH, 128), jnp.float32),
  )(edge_index)


def _proj_body(h_ref, wl_ref, wr_ref, b_ref, p_ref, q_ref):
  h = h_ref[...]
  p_ref[...] = jnp.dot(h, wl_ref[...], preferred_element_type=jnp.float32)
  q_ref[...] = (jnp.dot(h, wr_ref[...], preferred_element_type=jnp.float32)
                + b_ref[...])


def _proj(h, wlT, wrT, b):
  return pl.pallas_call(
      _proj_body,
      grid=(NP // _BM,),
      in_specs=[
          pl.BlockSpec((_BM, D), lambda i: (i, 0)),
          pl.BlockSpec((D, D), lambda i: (0, 0)),
          pl.BlockSpec((D, D), lambda i: (0, 0)),
          pl.BlockSpec((1, D), lambda i: (0, 0)),
      ],
      out_specs=[
          pl.BlockSpec((_BM, D), lambda i: (i, 0)),
          pl.BlockSpec((_BM, D), lambda i: (i, 0)),
      ],
      out_shape=[
          jax.ShapeDtypeStruct((NP, D), jnp.float32),
          jax.ShapeDtypeStruct((NP, D), jnp.float32),
      ],
  )(h, wlT, wrT, b)


def _combine_body(part_ref, cnt_ref, q_ref, o_ref, *, softmax):
  ssum = part_ref[0] + part_ref[1]
  c = cnt_ref[...]
  h = jnp.maximum(ssum / jnp.maximum(c, 1.0) + q_ref[...], 0.0)
  if softmax:
    m = jnp.max(h, axis=1, keepdims=True)
    lse = jnp.log(jnp.sum(jnp.exp(h - m), axis=1, keepdims=True)) + m
    h = h - lse
  o_ref[...] = h


def _combine(part, cnt, q, softmax):
  nrows = N if softmax else NP
  return pl.pallas_call(
      functools.partial(_combine_body, softmax=softmax),
      grid=(NP // _BM,),
      in_specs=[
          pl.BlockSpec((2, _BM, D), lambda i: (0, i, 0)),
          pl.BlockSpec((_BM, 1), lambda i: (i, 0)),
          pl.BlockSpec((_BM, D), lambda i: (i, 0)),
      ],
      out_specs=pl.BlockSpec((_BM, D), lambda i: (i, 0)),
      out_shape=jax.ShapeDtypeStruct((nrows, D), jnp.float32),
  )(part, cnt, q)


def kernel(x, edge_index, W1l, b1, W1r, W2l, b2, W2r):
  src = edge_index[0]
  dst = edge_index[1]

  cnt = _count(edge_index).reshape(NP, 1)

  p1, q1 = _proj(x, W1l.T, W1r.T, b1[None])
  acc1 = _edge_pass(p1, src, dst).reshape(NC, NP, D)
  h1 = _combine(acc1, cnt, q1, softmax=False)

  p2, q2 = _proj(h1, W2l.T, W2r.T, b2[None])
  acc2 = _edge_pass(p2, src, dst).reshape(NC, NP, D)
  h2 = _combine(acc2, cnt, q2, softmax=True)
  return h2


# speedup vs baseline: 14.5950x; 1.0003x over previous
import functools

import jax
import jax.numpy as jnp
from jax import lax
from jax.experimental import pallas as pl
from jax.experimental.pallas import tpu as pltpu
from jax.experimental.pallas import tpu_sc as plsc

N = 10000
E = 320000
D = 128
NP = 10240
NC = 2
NS = 16
NW = NC * NS
EPW = E // NW
CH = 80
NCH = EPW // CH
RPS = NP // NS


NB = 4
GD = 2
NBI = 6


def _edge_body(p_hbm, src_hbm, dst_hbm, out_hbm, ei_v, rows_v, acc_sh, sem_i,
               sem_g, sem_s):
  c = lax.axis_index("c")
  s = lax.axis_index("s")
  wid = c * NS + s
  base = wid * EPW

  zeros16 = jnp.zeros((16,), jnp.float32)

  def zero_row(i, _):
    for j in range(D // 16):
      rows_v[0, i, pl.ds(j * 16, 16)] = zeros16
    return 0

  lax.fori_loop(0, CH, zero_row, 0)
  row0 = s * RPS
  for i in range(RPS // CH):
    pltpu.sync_copy(rows_v.at[0], acc_sh.at[pl.ds(row0 + i * CH, CH)])
  plsc.subcore_barrier()

  def idx_descs(f):
    start = pl.multiple_of(base + f * CH, 8)
    bi = f % NBI
    return (
        pltpu.make_async_copy(src_hbm.at[pl.ds(start, CH)], ei_v.at[bi, 0],
                              sem_i.at[bi]),
        pltpu.make_async_copy(dst_hbm.at[pl.ds(start, CH)], ei_v.at[bi, 1],
                              sem_i.at[bi]),
    )

  def idx_start(f):
    for dsc in idx_descs(f):
      dsc.start()

  def idx_wait(f):
    for dsc in idx_descs(f):
      dsc.wait()

  def gather_start(f):
    b = f % NB
    pltpu.async_copy(p_hbm.at[ei_v.at[f % NBI, 0]], rows_v.at[b],
                     sem_g.at[b])

  def gather_wait(j):
    b = j % NB
    pltpu.make_async_copy(p_hbm.at[ei_v.at[j % NBI, 0]], rows_v.at[b],
                          sem_g.at[b]).wait()

  def scatter_start(j):
    b = j % NB
    pltpu.async_copy(rows_v.at[b], acc_sh.at[ei_v.at[j % NBI, 1]],
                     sem_s.at[b], add=True)

  def scatter_wait(j):
    b = j % NB
    pltpu.make_async_copy(rows_v.at[b], acc_sh.at[ei_v.at[j % NBI, 1]],
                          sem_s.at[b]).wait()

  for f in range(GD + 1):
    idx_start(f)
  for f in range(GD):
    idx_wait(f)
    gather_start(f)

  def chunk(j, _):
    gather_wait(j)
    scatter_start(j)
    f = j + GD

    @pl.when(f < NCH)
    def _():
      idx_wait(f)
      @pl.when(f >= NB)
      def _():
        scatter_wait(f - NB)

      gather_start(f)

    fx = j + GD + 1

    @pl.when(fx < NCH)
    def _():
      idx_start(fx)

    return 0

  lax.fori_loop(0, NCH, chunk, 0)
  for j in range(NCH - NB, NCH):
    scatter_wait(j)
  plsc.subcore_barrier()

  out_row = c * NP + s * RPS
  pltpu.sync_copy(acc_sh.at[pl.ds(s * RPS, RPS)],
                  out_hbm.at[pl.ds(out_row, RPS)])


_edge_pass = pl.kernel(
    _edge_body,
    out_type=jax.ShapeDtypeStruct((NC * NP, D), jnp.float32),
    mesh=plsc.VectorSubcoreMesh(core_axis_name="c", subcore_axis_name="s"),
    scratch_types=[
        pltpu.VMEM((NBI, 2, CH), jnp.int32),
        pltpu.VMEM((NB, CH, D), jnp.float32),
        pltpu.VMEM_SHARED((NP, D), jnp.float32),
        pltpu.SemaphoreType.DMA((NBI,)),
        pltpu.SemaphoreType.DMA((NB,)),
        pltpu.SemaphoreType.DMA((NB,)),
    ],
)


_BM = 2048
_EK = 6400
_NH = NP // 128


def _count_body(dst_ref, cnt_ref):
  d = dst_ref[1:2, :]
  hi = d // 128
  lo = d - hi * 128
  ihi = lax.broadcasted_iota(jnp.int32, (_NH, _EK), 0)
  ilo = lax.broadcasted_iota(jnp.int32, (128, _EK), 0)
  oh_hi = jnp.where(hi == ihi, 1.0, 0.0).astype(jnp.bfloat16)
  oh_lo = jnp.where(lo == ilo, 1.0, 0.0).astype(jnp.bfloat16)
  part = lax.dot_general(oh_hi, oh_lo, (((1,), (1,)), ((), ())),
                         preferred_element_type=jnp.float32)

  @pl.when(pl.program_id(0) == 0)
  def _():
    cnt_ref[...] = jnp.zeros_like(cnt_ref)

  cnt_ref[...] += part


def _count(edge_index):
  return pl.pallas_call(
      _count_body,
      grid=(E // _EK,),
      in_specs=[pl.BlockSpec((2, _EK), lambda i: (0, i))],
      out_specs=pl.BlockSpec((_NH, 128), lambda i: (0, 0)),
      out_shape=jax.ShapeDtypeStruct((_NH, 128), jnp.float32),
  )(edge_index)


def _proj_body(h_ref, wl_ref, wr_ref, b_ref, p_ref, q_ref):
  h = h_ref[...]
  p_ref[...] = jnp.dot(h, wl_ref[...], preferred_element_type=jnp.float32)
  q_ref[...] = (jnp.dot(h, wr_ref[...], preferred_element_type=jnp.float32)
                + b_ref[...])


def _proj(h, wlT, wrT, b):
  return pl.pallas_call(
      _proj_body,
      grid=(NP // _BM,),
      in_specs=[
          pl.BlockSpec((_BM, D), lambda i: (i, 0)),
          pl.BlockSpec((D, D), lambda i: (0, 0)),
          pl.BlockSpec((D, D), lambda i: (0, 0)),
          pl.BlockSpec((1, D), lambda i: (0, 0)),
      ],
      out_specs=[
          pl.BlockSpec((_BM, D), lambda i: (i, 0)),
          pl.BlockSpec((_BM, D), lambda i: (i, 0)),
      ],
      out_shape=[
          jax.ShapeDtypeStruct((NP, D), jnp.float32),
          jax.ShapeDtypeStruct((NP, D), jnp.float32),
      ],
  )(h, wlT, wrT, b)


def _combine_body(part_ref, cnt_ref, q_ref, o_ref, *, softmax):
  ssum = part_ref[0] + part_ref[1]
  c = cnt_ref[...]
  h = jnp.maximum(ssum / jnp.maximum(c, 1.0) + q_ref[...], 0.0)
  if softmax:
    m = jnp.max(h, axis=1, keepdims=True)
    lse = jnp.log(jnp.sum(jnp.exp(h - m), axis=1, keepdims=True)) + m
    h = h - lse
  o_ref[...] = h


def _combine(part, cnt, q, softmax):
  nrows = N if softmax else NP
  return pl.pallas_call(
      functools.partial(_combine_body, softmax=softmax),
      grid=(NP // _BM,),
      in_specs=[
          pl.BlockSpec((2, _BM, D), lambda i: (0, i, 0)),
          pl.BlockSpec((_BM, 1), lambda i: (i, 0)),
          pl.BlockSpec((_BM, D), lambda i: (i, 0)),
      ],
      out_specs=pl.BlockSpec((_BM, D), lambda i: (i, 0)),
      out_shape=jax.ShapeDtypeStruct((nrows, D), jnp.float32),
  )(part, cnt, q)


def kernel(x, edge_index, W1l, b1, W1r, W2l, b2, W2r):
  src = edge_index[0]
  dst = edge_index[1]

  p1, q1 = _proj(x, W1l.T, W1r.T, b1[None])
  acc1 = _edge_pass(p1, src, dst).reshape(NC, NP, D)
  cnt = _count(edge_index).reshape(NP, 1)
  h1 = _combine(acc1, cnt, q1, softmax=False)

  p2, q2 = _proj(h1, W2l.T, W2r.T, b2[None])
  acc2 = _edge_pass(p2, src, dst).reshape(NC, NP, D)
  h2 = _combine(acc2, cnt, q2, softmax=True)
  return h2


# speedup vs baseline: 14.8719x; 1.0190x over previous
import functools

import jax
import jax.numpy as jnp
from jax import lax
from jax.experimental import pallas as pl
from jax.experimental.pallas import tpu as pltpu
from jax.experimental.pallas import tpu_sc as plsc

N = 10000
E = 320000
D = 128
NP = 10240
NC = 2
NS = 16
NW = NC * NS
EPW = E // NW
CH = 80
NCH = EPW // CH
RPS = NP // NS


NB = 4
GD = 2
NBI = 6


def _edge_body(p_hbm, src_hbm, dst_hbm, out_hbm, ei_v, rows_v, acc_sh, sem_i,
               sem_g, sem_s):
  c = lax.axis_index("c")
  s = lax.axis_index("s")
  wid = c * NS + s
  base = wid * EPW

  zeros16 = jnp.zeros((16,), jnp.float32)

  def zero_row(i, _):
    for j in range(D // 16):
      rows_v[0, i, pl.ds(j * 16, 16)] = zeros16
    return 0

  lax.fori_loop(0, CH, zero_row, 0)
  row0 = s * RPS
  for i in range(RPS // CH):
    pltpu.sync_copy(rows_v.at[0], acc_sh.at[pl.ds(row0 + i * CH, CH)])
  plsc.subcore_barrier()

  def idx_descs(f):
    start = pl.multiple_of(base + f * CH, 8)
    bi = f % NBI
    return (
        pltpu.make_async_copy(src_hbm.at[pl.ds(start, CH)], ei_v.at[bi, 0],
                              sem_i.at[bi]),
        pltpu.make_async_copy(dst_hbm.at[pl.ds(start, CH)], ei_v.at[bi, 1],
                              sem_i.at[bi]),
    )

  def idx_start(f):
    for dsc in idx_descs(f):
      dsc.start()

  def idx_wait(f):
    for dsc in idx_descs(f):
      dsc.wait()

  def gather_start(f):
    b = f % NB
    pltpu.async_copy(p_hbm.at[ei_v.at[f % NBI, 0]], rows_v.at[b],
                     sem_g.at[b])

  def gather_wait(j):
    b = j % NB
    pltpu.make_async_copy(p_hbm.at[ei_v.at[j % NBI, 0]], rows_v.at[b],
                          sem_g.at[b]).wait()

  def scatter_start(j):
    b = j % NB
    pltpu.async_copy(rows_v.at[b], acc_sh.at[ei_v.at[j % NBI, 1]],
                     sem_s.at[b], add=True)

  def scatter_wait(j):
    b = j % NB
    pltpu.make_async_copy(rows_v.at[b], acc_sh.at[ei_v.at[j % NBI, 1]],
                          sem_s.at[b]).wait()

  for f in range(GD + 1):
    idx_start(f)
  for f in range(GD):
    idx_wait(f)
    gather_start(f)

  def chunk(j, _):
    gather_wait(j)
    scatter_start(j)
    f = j + GD

    @pl.when(f < NCH)
    def _():
      idx_wait(f)
      @pl.when(f >= NB)
      def _():
        scatter_wait(f - NB)

      gather_start(f)

    fx = j + GD + 1

    @pl.when(fx < NCH)
    def _():
      idx_start(fx)

    return 0

  lax.fori_loop(0, NCH, chunk, 0)
  for j in range(NCH - NB, NCH):
    scatter_wait(j)
  plsc.subcore_barrier()

  out_row = c * NP + s * RPS
  pltpu.sync_copy(acc_sh.at[pl.ds(s * RPS, RPS)],
                  out_hbm.at[pl.ds(out_row, RPS)])


_edge_pass = pl.kernel(
    _edge_body,
    out_type=jax.ShapeDtypeStruct((NC * NP, D), jnp.float32),
    mesh=plsc.VectorSubcoreMesh(core_axis_name="c", subcore_axis_name="s"),
    scratch_types=[
        pltpu.VMEM((NBI, 2, CH), jnp.int32),
        pltpu.VMEM((NB, CH, D), jnp.float32),
        pltpu.VMEM_SHARED((NP, D), jnp.float32),
        pltpu.SemaphoreType.DMA((NBI,)),
        pltpu.SemaphoreType.DMA((NB,)),
        pltpu.SemaphoreType.DMA((NB,)),
    ],
)


_BM = 2048
_EK = 6400
_NH = NP // 128


def _count_body(dst_ref, cnt_ref):
  d = dst_ref[1:2, :]
  hi = d // 128
  lo = d - hi * 128
  ihi = lax.broadcasted_iota(jnp.int32, (_NH, _EK), 0)
  ilo = lax.broadcasted_iota(jnp.int32, (128, _EK), 0)
  oh_hi = jnp.where(hi == ihi, 1.0, 0.0).astype(jnp.bfloat16)
  oh_lo = jnp.where(lo == ilo, 1.0, 0.0).astype(jnp.bfloat16)
  part = lax.dot_general(oh_hi, oh_lo, (((1,), (1,)), ((), ())),
                         preferred_element_type=jnp.float32)

  @pl.when(pl.program_id(0) == 0)
  def _():
    cnt_ref[...] = jnp.zeros_like(cnt_ref)

  cnt_ref[...] += part


def _count(edge_index):
  return pl.pallas_call(
      _count_body,
      grid=(E // _EK,),
      in_specs=[pl.BlockSpec((2, _EK), lambda i: (0, i))],
      out_specs=pl.BlockSpec((_NH, 128), lambda i: (0, 0)),
      out_shape=jax.ShapeDtypeStruct((_NH, 128), jnp.float32),
  )(edge_index)


def _proj_body(h_ref, wl_ref, wr_ref, b_ref, p_ref, q_ref):
  h = h_ref[...]
  p_ref[...] = jnp.dot(h, wl_ref[...], preferred_element_type=jnp.float32)
  q_ref[...] = (jnp.dot(h, wr_ref[...], preferred_element_type=jnp.float32)
                + b_ref[...])


def _proj(h, wlT, wrT, b):
  return pl.pallas_call(
      _proj_body,
      grid=(NP // _BM,),
      in_specs=[
          pl.BlockSpec((_BM, D), lambda i: (i, 0)),
          pl.BlockSpec((D, D), lambda i: (0, 0)),
          pl.BlockSpec((D, D), lambda i: (0, 0)),
          pl.BlockSpec((1, D), lambda i: (0, 0)),
      ],
      out_specs=[
          pl.BlockSpec((_BM, D), lambda i: (i, 0)),
          pl.BlockSpec((_BM, D), lambda i: (i, 0)),
      ],
      out_shape=[
          jax.ShapeDtypeStruct((NP, D), jnp.float32),
          jax.ShapeDtypeStruct((NP, D), jnp.float32),
      ],
  )(h, wlT, wrT, b)


def _combine_proj_body(part_ref, cnt_ref, q_ref, wl_ref, wr_ref, b_ref,
                       p_ref, q2_ref):
  ssum = part_ref[0] + part_ref[1]
  c = cnt_ref[...]
  h = jnp.maximum(ssum / jnp.maximum(c, 1.0) + q_ref[...], 0.0)
  p_ref[...] = jnp.dot(h, wl_ref[...], preferred_element_type=jnp.float32)
  q2_ref[...] = (jnp.dot(h, wr_ref[...], preferred_element_type=jnp.float32)
                 + b_ref[...])


def _combine_proj(part, cnt, q, wlT, wrT, b):
  return pl.pallas_call(
      _combine_proj_body,
      grid=(NP // _BM,),
      in_specs=[
          pl.BlockSpec((2, _BM, D), lambda i: (0, i, 0)),
          pl.BlockSpec((_BM, 1), lambda i: (i, 0)),
          pl.BlockSpec((_BM, D), lambda i: (i, 0)),
          pl.BlockSpec((D, D), lambda i: (0, 0)),
          pl.BlockSpec((D, D), lambda i: (0, 0)),
          pl.BlockSpec((1, D), lambda i: (0, 0)),
      ],
      out_specs=[
          pl.BlockSpec((_BM, D), lambda i: (i, 0)),
          pl.BlockSpec((_BM, D), lambda i: (i, 0)),
      ],
      out_shape=[
          jax.ShapeDtypeStruct((NP, D), jnp.float32),
          jax.ShapeDtypeStruct((NP, D), jnp.float32),
      ],
  )(part, cnt, q, wlT, wrT, b)


def _combine_body(part_ref, cnt_ref, q_ref, o_ref, *, softmax):
  ssum = part_ref[0] + part_ref[1]
  c = cnt_ref[...]
  h = jnp.maximum(ssum / jnp.maximum(c, 1.0) + q_ref[...], 0.0)
  if softmax:
    m = jnp.max(h, axis=1, keepdims=True)
    lse = jnp.log(jnp.sum(jnp.exp(h - m), axis=1, keepdims=True)) + m
    h = h - lse
  o_ref[...] = h


def _combine(part, cnt, q, softmax):
  nrows = N if softmax else NP
  return pl.pallas_call(
      functools.partial(_combine_body, softmax=softmax),
      grid=(NP // _BM,),
      in_specs=[
          pl.BlockSpec((2, _BM, D), lambda i: (0, i, 0)),
          pl.BlockSpec((_BM, 1), lambda i: (i, 0)),
          pl.BlockSpec((_BM, D), lambda i: (i, 0)),
      ],
      out_specs=pl.BlockSpec((_BM, D), lambda i: (i, 0)),
      out_shape=jax.ShapeDtypeStruct((nrows, D), jnp.float32),
  )(part, cnt, q)


def kernel(x, edge_index, W1l, b1, W1r, W2l, b2, W2r):
  src = edge_index[0]
  dst = edge_index[1]

  p1, q1 = _proj(x, W1l.T, W1r.T, b1[None])
  acc1 = _edge_pass(p1, src, dst).reshape(NC, NP, D)
  cnt = _count(edge_index).reshape(NP, 1)
  p2, q2 = _combine_proj(acc1, cnt, q1, W2l.T, W2r.T, b2[None])
  acc2 = _edge_pass(p2, src, dst).reshape(NC, NP, D)
  h2 = _combine(acc2, cnt, q2, softmax=True)
  return h2
